# Initial kernel scaffold; baseline (speedup 1.0000x reference)
import jax
import jax.numpy as jnp
from jax.experimental import pallas as pl


def kernel(indices, tier0, tier1, tier2):
    raise NotImplementedError("write your pallas kernel here")


# speedup vs baseline: 2.1296x; 2.1296x over previous
import functools

import jax
import jax.numpy as jnp
from jax import lax
from jax.experimental import pallas as pl
from jax.experimental.pallas import tpu as pltpu
from jax.experimental.pallas import tpu_sc as plsc

EMBED_DIM = 128
NUM_ROWS = 256
BATCH = 16384
IDX_CHUNK = 128


@functools.cache
def _build_gather():
    info = plsc.get_sparse_core_info()
    num_cores, num_subcores, lanes = info.num_cores, info.num_subcores, info.num_lanes
    num_workers = num_cores * num_subcores
    b_per_w = BATCH // num_workers
    n_chunks = b_per_w // IDX_CHUNK
    mesh = plsc.VectorSubcoreMesh(core_axis_name="c", subcore_axis_name="s")

    @functools.partial(
        pl.kernel,
        mesh=mesh,
        out_type=jax.ShapeDtypeStruct((BATCH, EMBED_DIM), jnp.float32),
        scratch_types=[
            pltpu.VMEM((n_chunks, IDX_CHUNK), jnp.int32),
            pltpu.VMEM((b_per_w, EMBED_DIM), jnp.float32),
            pltpu.SemaphoreType.DMA,
        ],
    )
    def gather_kernel(table_hbm, idx_hbm, out_hbm, idx_v, rows_v, sem):
        wid = lax.axis_index("s") * num_cores + lax.axis_index("c")
        pltpu.sync_copy(idx_hbm.at[wid], idx_v)
        for j in range(n_chunks):
            for i in range(IDX_CHUNK // lanes):
                v = idx_v[j, pl.ds(i * lanes, lanes)]
                valid = (v >= 0) & (v < NUM_ROWS)
                idx_v[j, pl.ds(i * lanes, lanes)] = jnp.where(valid, v, NUM_ROWS)
        copies = [
            pltpu.async_copy(
                table_hbm.at[idx_v.at[j]],
                rows_v.at[pl.ds(j * IDX_CHUNK, IDX_CHUNK)],
                sem,
            )
            for j in range(n_chunks)
        ]
        for c in copies:
            c.wait()
        pltpu.sync_copy(rows_v, out_hbm.at[pl.ds(wid * b_per_w, b_per_w)])

    return gather_kernel, num_workers, n_chunks


def kernel(indices, tier0, tier1, tier2):
    gather, num_workers, n_chunks = _build_gather()
    table = jnp.concatenate(
        [tier0, tier1, tier2, jnp.zeros((1, EMBED_DIM), jnp.float32)], axis=0
    )
    idx = indices.astype(jnp.int32).reshape(num_workers, n_chunks, IDX_CHUNK)
    return gather(table, idx)
